# baseline (speedup 1.0000x reference)
import jax
import jax.numpy as jnp
from jax import lax
from jax.experimental import pallas as pl
from jax.experimental.pallas import tpu as pltpu
from jax.experimental.pallas import tpu_sc as plsc

NUM_USERS = 45000
NUM_ITEMS = 5000
N_NODES = NUM_USERS + NUM_ITEMS
EMB = 256
NE = 120000
GCN_K = 4
K = 2
ALPHA = 0.9
BETA = 0.1

NC, NS, L = 2, 16, 16
NW = NC * NS
R = 200
B = 128
UCH = NUM_USERS // R
ICH = NUM_ITEMS // R
TOT = UCH + ICH
SLOTS = (TOT + NW - 1) // NW
EPAD = NE + 4 * B

R2 = 200
VCH = NUM_ITEMS // R2
TOT2 = 2 * VCH
SLOTS2 = (TOT2 + NW - 1) // NW


def _ffn_bn_body(x_ref, w1_ref, b1_ref, w2_ref, b2_ref, g_ref, bb_ref, o_ref):
    x = x_ref[...]
    h = jnp.maximum(jnp.dot(x, w1_ref[...], preferred_element_type=jnp.float32)
                    + b1_ref[...], 0.0)
    h = jnp.dot(h, w2_ref[...], preferred_element_type=jnp.float32) + b2_ref[...]
    m = jnp.mean(h, axis=0, keepdims=True)
    v = jnp.mean((h - m) ** 2, axis=0, keepdims=True)
    o_ref[...] = (h - m) * jax.lax.rsqrt(v + 1e-05) * g_ref[...] + bb_ref[...]


def _ffn_bn(x, W1, b1, W2, b2, g, bb):
    n = x.shape[0]
    return pl.pallas_call(
        _ffn_bn_body,
        out_shape=jax.ShapeDtypeStruct((n, EMB), jnp.float32),
    )(x, W1, b1.reshape(1, -1), W2, b2.reshape(1, -1),
      g.reshape(1, -1), bb.reshape(1, -1))


def _sc_scalar(ref, i):
    return ref[pl.ds(i, L)][0]


def _zero_rows(ref, n):
    z = jnp.zeros((L,), jnp.float32)

    def body(r, _):
        for t in range(EMB // L):
            ref[r, pl.ds(t * L, L)] = z
        return 0
    lax.fori_loop(0, n, body, 0, unroll=4)


def _edge_blocks(e0, e1, row0, tab_src, e_src, e_dst, e_w,
                 dst_v, buf0, buf1, drain_v, rmax, iota):
    eb0 = (e0 // 8) * 8
    nb = jnp.maximum((e1 - eb0 + B - 1) // B, 0)
    npair = (nb + 1) // 2

    def load(off, idx_v, w_v, lidx_v):
        pltpu.sync_copy(e_src.at[pl.ds(off, B)], idx_v)
        pltpu.sync_copy(e_dst.at[pl.ds(off, B)], dst_v)
        pltpu.sync_copy(e_w.at[pl.ds(off, B)], w_v.at[pl.ds(0, B)])
        for t in range(B // L):
            sl = pl.ds(t * L, L)
            ev = off + t * L + iota
            m = (ev >= e0) & (ev < e1)
            w_v[sl] = jnp.where(m, w_v[sl], 0.0)
            lidx_v[sl] = jnp.clip(dst_v[sl] - row0, 0, rmax - 1)

    def accum(w_v, lidx_v, rows_v):
        def body(r, _):
            wsp = _sc_scalar(w_v, r)
            di = _sc_scalar(lidx_v, r)
            for t in range(EMB // L):
                sl = pl.ds(t * L, L)
                drain_v[di, sl] = drain_v[di, sl] + rows_v[r, sl] * wsp
            return 0
        lax.fori_loop(0, B, body, 0, unroll=4)

    i0, w0, l0, r0, s0 = buf0
    i1, w1, l1, r1, s1 = buf1
    load(eb0, i0, w0, l0)
    pltpu.async_copy(tab_src.at[i0], r0, s0)

    def pair(j, _):
        off = eb0 + 2 * j * B
        load(off + B, i1, w1, l1)
        pltpu.async_copy(tab_src.at[i1], r1, s1)
        pltpu.make_async_copy(tab_src.at[i0], r0, s0).wait()
        accum(w0, l0, r0)
        load(off + 2 * B, i0, w0, l0)
        pltpu.async_copy(tab_src.at[i0], r0, s0)
        pltpu.make_async_copy(tab_src.at[i1], r1, s1).wait()
        accum(w1, l1, r1)
        return 0

    lax.fori_loop(0, npair, pair, 0, unroll=False)
    pltpu.make_async_copy(tab_src.at[i0], r0, s0).wait()


def _prop_body(hu, hi, su, du, wu, bu, si, di, wi, bi, invd,
               out_u, out_i,
               bu_v, bi_v, dst_v, idx0, w0, l0, rows0, sem0,
               idx1, w1, l1, rows1, sem1, drain_v, invd_v):
    c = lax.axis_index("c")
    s = lax.axis_index("s")
    wid = c * NS + s
    buf0 = (idx0, w0, l0, rows0, sem0)
    buf1 = (idx1, w1, l1, rows1, sem1)
    rows_v = rows0
    pltpu.sync_copy(bu, bu_v)
    pltpu.sync_copy(bi, bi_v)
    iota = jax.lax.iota(jnp.int32, L)

    def do_chunk(e0, e1, row0, tab_src, tab_dst, e_src, e_dst, e_w, ioff, out):
        _zero_rows(drain_v, R)
        _edge_blocks(e0, e1, row0, tab_src, e_src, e_dst, e_w,
                     dst_v, buf0, buf1, drain_v, R, iota)
        pltpu.sync_copy(invd.at[pl.ds(ioff + row0, R)], invd_v.at[pl.ds(0, R)])

        def diag_piece(base, n):
            pltpu.sync_copy(tab_dst.at[pl.ds(row0 + base, n)],
                            rows_v.at[pl.ds(0, n)])

            def body(r, _):
                dsp = _sc_scalar(invd_v, base + r)
                for t in range(EMB // L):
                    sl = pl.ds(t * L, L)
                    drain_v[base + r, sl] = (drain_v[base + r, sl]
                                             + rows_v[r, sl] * dsp)
                return 0
            lax.fori_loop(0, n, body, 0, unroll=False)

        diag_piece(0, B)
        diag_piece(B, R - B)
        pltpu.sync_copy(drain_v, out.at[pl.ds(row0, R)])

    def slot(k, _):
        cid = k * NW + wid

        @pl.when(cid < UCH)
        def _():
            bv16 = bu_v[pl.ds(cid, L)]
            do_chunk(bv16[0], bv16[1], cid * R, hi, hu, su, du, wu, 0, out_u)

        @pl.when((cid >= UCH) & (cid < TOT))
        def _():
            j = cid - UCH
            bv16 = bi_v[pl.ds(j, L)]
            do_chunk(bv16[0], bv16[1], j * R, hu, hi, si, di, wi,
                     NUM_USERS, out_i)

        return 0

    lax.fori_loop(0, SLOTS, slot, 0, unroll=False)


def _make_prop():
    mesh = plsc.VectorSubcoreMesh(core_axis_name="c", subcore_axis_name="s",
                                  num_cores=NC, num_subcores=NS)
    return pl.kernel(
        _prop_body,
        out_type=(jax.ShapeDtypeStruct((NUM_USERS, EMB), jnp.float32),
                  jax.ShapeDtypeStruct((NUM_ITEMS, EMB), jnp.float32)),
        mesh=mesh,
        scratch_types=[
            pltpu.VMEM((240,), jnp.int32),
            pltpu.VMEM((40,), jnp.int32),
            pltpu.VMEM((B,), jnp.int32),
            pltpu.VMEM((B,), jnp.int32),
            pltpu.VMEM((B + L,), jnp.float32),
            pltpu.VMEM((B + L,), jnp.int32),
            pltpu.VMEM((B, EMB), jnp.float32),
            pltpu.SemaphoreType.DMA,
            pltpu.VMEM((B,), jnp.int32),
            pltpu.VMEM((B + L,), jnp.float32),
            pltpu.VMEM((B + L,), jnp.int32),
            pltpu.VMEM((B, EMB), jnp.float32),
            pltpu.SemaphoreType.DMA,
            pltpu.VMEM((R, EMB), jnp.float32),
            pltpu.VMEM((R + L,), jnp.float32),
        ],
    )


def _prop_setup(g_src, g_dst, g_w):
    u = g_src[:NE].astype(jnp.int32)
    itm = (g_dst[:NE] - NUM_USERS).astype(jnp.int32)
    w = g_w[:NE]
    invd = g_w[2 * NE:]
    pu = jnp.argsort(u)
    du_s, su_s, wu_s = u[pu], itm[pu], w[pu]
    pi = jnp.argsort(itm)
    di_s, si_s, wi_s = itm[pi], u[pi], w[pi]
    bu = jnp.searchsorted(du_s, jnp.arange(0, NUM_USERS + 1, R)).astype(jnp.int32)
    bi = jnp.searchsorted(di_s, jnp.arange(0, NUM_ITEMS + 1, R)).astype(jnp.int32)
    bu = jnp.pad(bu, (0, 240 - UCH - 1))
    bi = jnp.pad(bi, (0, 40 - ICH - 1))
    pad = EPAD - NE
    return (jnp.pad(su_s, (0, pad)), jnp.pad(du_s, (0, pad)),
            jnp.pad(wu_s, (0, pad)), bu,
            jnp.pad(si_s, (0, pad)), jnp.pad(di_s, (0, pad)),
            jnp.pad(wi_s, (0, pad)), bi, invd)


def _ii_body(zv, zt, xv, xt, sv, dv, wv, bv, st_, dt_, wt_, bt,
             out_v, out_t,
             bv_v, bt_v, dst_v, idx0, w0, l0, rows0, sem0,
             idx1, w1, l1, rows1, sem1, drain_v):
    c = lax.axis_index("c")
    s = lax.axis_index("s")
    wid = c * NS + s
    buf0 = (idx0, w0, l0, rows0, sem0)
    buf1 = (idx1, w1, l1, rows1, sem1)
    rows_v = rows0
    pltpu.sync_copy(bv, bv_v)
    pltpu.sync_copy(bt, bt_v)
    iota = jax.lax.iota(jnp.int32, L)

    def do_chunk(e0, e1, row0, ztab, xtab, e_src, e_dst, e_w, out):
        _zero_rows(drain_v, R2)
        _edge_blocks(e0, e1, row0, ztab, e_src, e_dst, e_w,
                     dst_v, buf0, buf1, drain_v, R2, iota)

        def mix_piece(base, n):
            pltpu.sync_copy(xtab.at[pl.ds(row0 + base, n)],
                            rows_v.at[pl.ds(0, n)])

            def body(r, _):
                for t in range(EMB // L):
                    sl = pl.ds(t * L, L)
                    drain_v[base + r, sl] = (ALPHA * drain_v[base + r, sl]
                                             + BETA * rows_v[r, sl])
                return 0
            lax.fori_loop(0, n, body, 0, unroll=False)

        mix_piece(0, B)
        mix_piece(B, R2 - B)
        pltpu.sync_copy(drain_v, out.at[pl.ds(row0, R2)])

    def slot(k, _):
        cid = k * NW + wid

        @pl.when(cid < VCH)
        def _():
            bv16 = bv_v[pl.ds(cid, L)]
            do_chunk(bv16[0], bv16[1], cid * R2, zv, xv, sv, dv, wv, out_v)

        @pl.when((cid >= VCH) & (cid < TOT2))
        def _():
            j = cid - VCH
            bv16 = bt_v[pl.ds(j, L)]
            do_chunk(bv16[0], bv16[1], j * R2, zt, xt, st_, dt_, wt_, out_t)

        return 0

    lax.fori_loop(0, SLOTS2, slot, 0, unroll=False)


def _make_ii():
    mesh = plsc.VectorSubcoreMesh(core_axis_name="c", subcore_axis_name="s",
                                  num_cores=NC, num_subcores=NS)
    return pl.kernel(
        _ii_body,
        out_type=(jax.ShapeDtypeStruct((NUM_ITEMS, EMB), jnp.float32),
                  jax.ShapeDtypeStruct((NUM_ITEMS, EMB), jnp.float32)),
        mesh=mesh,
        scratch_types=[
            pltpu.VMEM((40,), jnp.int32),
            pltpu.VMEM((40,), jnp.int32),
            pltpu.VMEM((B,), jnp.int32),
            pltpu.VMEM((B,), jnp.int32),
            pltpu.VMEM((B + L,), jnp.float32),
            pltpu.VMEM((B + L,), jnp.int32),
            pltpu.VMEM((B, EMB), jnp.float32),
            pltpu.SemaphoreType.DMA,
            pltpu.VMEM((B,), jnp.int32),
            pltpu.VMEM((B + L,), jnp.float32),
            pltpu.VMEM((B + L,), jnp.int32),
            pltpu.VMEM((B, EMB), jnp.float32),
            pltpu.SemaphoreType.DMA,
            pltpu.VMEM((R2, EMB), jnp.float32),
        ],
    )


def _ii_setup(src, dst, w):
    src32 = src.astype(jnp.int32)
    dst32 = dst.astype(jnp.int32)
    b = jnp.searchsorted(
        dst32, jnp.arange(0, NUM_ITEMS + 1, R2, dtype=jnp.int32)).astype(jnp.int32)
    b = jnp.pad(b, (0, 40 - VCH - 1))
    return (jnp.pad(src32, (0, 4 * B)), jnp.pad(dst32, (0, 4 * B)),
            jnp.pad(w, (0, 4 * B)), b)


def _combine_body(ue_ref, ie_ref, h1u_ref, h1i_ref, h2u_ref, h2i_ref,
                  vh_ref, th_ref, o_ref):
    i = pl.program_id(0)
    inv = 1.0 / (K + 1)

    @pl.when(i < NUM_USERS // 1000)
    def _():
        o_ref[...] = (ue_ref[...] + h1u_ref[...] + h2u_ref[...]) * inv

    @pl.when(i >= NUM_USERS // 1000)
    def _():
        o_ref[...] = ((ie_ref[...] + h1i_ref[...] + h2i_ref[...]) * inv
                      + vh_ref[...] + th_ref[...])


def kernel(user_emb, item_emb, v_feat, t_feat, vW1, vb1, vW2, vb2, vg, vbb,
           tW1, tb1, tW2, tb2, tg, tbb, v_src, v_dst, v_w,
           t_src, t_dst, t_w, g_src, g_dst, g_w):
    encode_v = _ffn_bn(v_feat, vW1, vb1, vW2, vb2, vg, vbb)
    encode_t = _ffn_bn(t_feat, tW1, tb1, tW2, tb2, tg, tbb)

    ii_prop = _make_ii()
    iargs = _ii_setup(v_src, v_dst, v_w) + _ii_setup(t_src, t_dst, t_w)
    zv, zt = encode_v, encode_t
    for _ in range(GCN_K):
        zv, zt = ii_prop(zv, zt, encode_v, encode_t, *iargs)

    prop = _make_prop()
    pargs = _prop_setup(g_src, g_dst, g_w)
    h1u, h1i = prop(user_emb, item_emb, *pargs)
    h2u, h2i = prop(h1u, h1i, *pargs)

    nu_b = NUM_USERS // 1000
    node_h = pl.pallas_call(
        _combine_body,
        grid=(N_NODES // 1000,),
        in_specs=[
            pl.BlockSpec((1000, EMB), lambda i: (jnp.minimum(i, nu_b - 1), 0)),
            pl.BlockSpec((1000, EMB), lambda i: (jnp.maximum(i - nu_b, 0), 0)),
            pl.BlockSpec((1000, EMB), lambda i: (jnp.minimum(i, nu_b - 1), 0)),
            pl.BlockSpec((1000, EMB), lambda i: (jnp.maximum(i - nu_b, 0), 0)),
            pl.BlockSpec((1000, EMB), lambda i: (jnp.minimum(i, nu_b - 1), 0)),
            pl.BlockSpec((1000, EMB), lambda i: (jnp.maximum(i - nu_b, 0), 0)),
            pl.BlockSpec((1000, EMB), lambda i: (jnp.maximum(i - nu_b, 0), 0)),
            pl.BlockSpec((1000, EMB), lambda i: (jnp.maximum(i - nu_b, 0), 0)),
        ],
        out_specs=pl.BlockSpec((1000, EMB), lambda i: (i, 0)),
        out_shape=jax.ShapeDtypeStruct((N_NODES, EMB), jnp.float32),
    )(user_emb, item_emb, h1u, h1i, h2u, h2i, zv, zt)
    return node_h


# speedup vs baseline: 1.1706x; 1.1706x over previous
import jax
import jax.numpy as jnp
from jax import lax
from jax.experimental import pallas as pl
from jax.experimental.pallas import tpu as pltpu
from jax.experimental.pallas import tpu_sc as plsc

NUM_USERS = 45000
NUM_ITEMS = 5000
N_NODES = NUM_USERS + NUM_ITEMS
EMB = 256
NE = 120000
GCN_K = 4
K = 2
ALPHA = 0.9
BETA = 0.1

NC, NS, L = 2, 16, 16
NW = NC * NS
R = 200
B = 128
UCH = NUM_USERS // R
ICH = NUM_ITEMS // R
TOT = UCH + ICH
SLOTS = (TOT + NW - 1) // NW
EPAD = NE + 4 * B

R2 = 200
VCH = NUM_ITEMS // R2
TOT2 = 2 * VCH
SLOTS2 = (TOT2 + NW - 1) // NW


def _ffn_bn_body(x_ref, w1_ref, b1_ref, w2_ref, b2_ref, g_ref, bb_ref, o_ref):
    x = x_ref[...]
    h = jnp.maximum(jnp.dot(x, w1_ref[...], preferred_element_type=jnp.float32)
                    + b1_ref[...], 0.0)
    h = jnp.dot(h, w2_ref[...], preferred_element_type=jnp.float32) + b2_ref[...]
    m = jnp.mean(h, axis=0, keepdims=True)
    v = jnp.mean((h - m) ** 2, axis=0, keepdims=True)
    o_ref[...] = (h - m) * jax.lax.rsqrt(v + 1e-05) * g_ref[...] + bb_ref[...]


def _ffn_bn(x, W1, b1, W2, b2, g, bb):
    n = x.shape[0]
    return pl.pallas_call(
        _ffn_bn_body,
        out_shape=jax.ShapeDtypeStruct((n, EMB), jnp.float32),
    )(x, W1, b1.reshape(1, -1), W2, b2.reshape(1, -1),
      g.reshape(1, -1), bb.reshape(1, -1))


def _sc_scalar(ref, i):
    return ref[pl.ds(i, L)][0]


def _zero_rows(ref, n):
    z = jnp.zeros((L,), jnp.float32)

    def body(r, _):
        for t in range(EMB // L):
            ref[r, pl.ds(t * L, L)] = z
        return 0
    lax.fori_loop(0, n, body, 0, unroll=4)


def _edge_blocks(e0, e1, row0, tab_src, e_src, e_dst, e_w,
                 dst_v, buf0, buf1, drain_v, rmax, iota):
    eb0 = (e0 // 8) * 8
    nb = jnp.maximum((e1 - eb0 + B - 1) // B, 0)
    npair = (nb + 1) // 2

    def load(off, idx_v, w_v, lidx_v):
        pltpu.sync_copy(e_src.at[pl.ds(off, B)], idx_v)
        pltpu.sync_copy(e_dst.at[pl.ds(off, B)], dst_v)
        pltpu.sync_copy(e_w.at[pl.ds(off, B)], w_v.at[pl.ds(0, B)])
        for t in range(B // L):
            sl = pl.ds(t * L, L)
            ev = off + t * L + iota
            m = (ev >= e0) & (ev < e1)
            w_v[sl] = jnp.where(m, w_v[sl], 0.0)
            lidx_v[sl] = jnp.clip(dst_v[sl] - row0, 0, rmax - 1)

    def accum(w_v, lidx_v, rows_v):
        def body(r, _):
            wsp = _sc_scalar(w_v, r)
            di = _sc_scalar(lidx_v, r)
            for t in range(EMB // L):
                sl = pl.ds(t * L, L)
                drain_v[di, sl] = drain_v[di, sl] + rows_v[r, sl] * wsp
            return 0
        lax.fori_loop(0, B, body, 0, unroll=4)

    i0, w0, l0, r0, s0 = buf0
    i1, w1, l1, r1, s1 = buf1
    load(eb0, i0, w0, l0)
    pltpu.async_copy(tab_src.at[i0], r0, s0)

    def pair(j, _):
        off = eb0 + 2 * j * B
        load(off + B, i1, w1, l1)
        pltpu.async_copy(tab_src.at[i1], r1, s1)
        pltpu.make_async_copy(tab_src.at[i0], r0, s0).wait()
        accum(w0, l0, r0)
        load(off + 2 * B, i0, w0, l0)
        pltpu.async_copy(tab_src.at[i0], r0, s0)
        pltpu.make_async_copy(tab_src.at[i1], r1, s1).wait()
        accum(w1, l1, r1)
        return 0

    lax.fori_loop(0, npair, pair, 0, unroll=False)
    pltpu.make_async_copy(tab_src.at[i0], r0, s0).wait()


def _prop_body(zu, zi, su, du, bu, si, di, bi,
               out_u, out_i,
               bu_v, bi_v, idx_v, dst_v, lidx_v, rows_v, drain_sh):
    c = lax.axis_index("c")
    s = lax.axis_index("s")
    wid = c * NS + s
    dump = R
    pltpu.sync_copy(bu, bu_v)
    pltpu.sync_copy(bi, bi_v)
    iota = jax.lax.iota(jnp.int32, L)

    def do_chunk(e0, e1, row0, tab_src, tab_own, e_src, e_dst, out):
        pltpu.sync_copy(tab_own.at[pl.ds(row0, R)],
                        drain_sh.at[pl.ds(0, R)])
        eb0 = (e0 // 8) * 8
        nb = jnp.maximum((e1 - eb0 + B - 1) // B, 0)

        def blk(j, _):
            off = eb0 + j * B
            pltpu.sync_copy(e_src.at[pl.ds(off, B)], idx_v)
            pltpu.sync_copy(e_dst.at[pl.ds(off, B)], dst_v)
            for t in range(B // L):
                sl = pl.ds(t * L, L)
                ev = off + t * L + iota
                m = (ev >= e0) & (ev < e1)
                li = jnp.clip(dst_v[sl] - row0, 0, R - 1)
                lidx_v[sl] = jnp.where(m, li, dump)
            pltpu.sync_copy(tab_src.at[idx_v], rows_v)

            def acc(r, _):
                di = _sc_scalar(lidx_v, r)
                for t in range(EMB // L):
                    asl = pl.ds(t * L, L)
                    drain_sh[di, asl] = drain_sh[di, asl] + rows_v[r, asl]
                return 0

            lax.fori_loop(0, B, acc, 0, unroll=4)
            return 0

        lax.fori_loop(0, nb, blk, 0, unroll=False)
        pltpu.sync_copy(drain_sh.at[pl.ds(0, R)], out.at[pl.ds(row0, R)])

    def slot(k, _):
        cid = k * NW + wid

        @pl.when(cid < UCH)
        def _():
            bv16 = bu_v[pl.ds(cid, L)]
            do_chunk(bv16[0], bv16[1], cid * R, zi, zu, su, du, out_u)

        @pl.when((cid >= UCH) & (cid < TOT))
        def _():
            j = cid - UCH
            bv16 = bi_v[pl.ds(j, L)]
            do_chunk(bv16[0], bv16[1], j * R, zu, zi, si, di, out_i)

        return 0

    lax.fori_loop(0, SLOTS, slot, 0, unroll=False)


def _make_prop():
    mesh = plsc.VectorSubcoreMesh(core_axis_name="c", subcore_axis_name="s",
                                  num_cores=NC, num_subcores=NS)
    return pl.kernel(
        _prop_body,
        out_type=(jax.ShapeDtypeStruct((NUM_USERS, EMB), jnp.float32),
                  jax.ShapeDtypeStruct((NUM_ITEMS, EMB), jnp.float32)),
        mesh=mesh,
        scratch_types=[
            pltpu.VMEM((240,), jnp.int32),
            pltpu.VMEM((40,), jnp.int32),
            pltpu.VMEM((B,), jnp.int32),
            pltpu.VMEM((B,), jnp.int32),
            pltpu.VMEM((B + L,), jnp.int32),
            pltpu.VMEM((B, EMB), jnp.float32),
            pltpu.VMEM((R + 8, EMB), jnp.float32),
        ],
    )


def _prop_setup(g_src, g_dst, g_w):
    u = g_src[:NE].astype(jnp.int32)
    itm = (g_dst[:NE] - NUM_USERS).astype(jnp.int32)
    invd = g_w[2 * NE:]
    pu = jnp.argsort(u)
    du_s, su_s = u[pu], itm[pu]
    pi = jnp.argsort(itm)
    di_s, si_s = itm[pi], u[pi]
    bu = jnp.searchsorted(du_s, jnp.arange(0, NUM_USERS + 1, R)).astype(jnp.int32)
    bi = jnp.searchsorted(di_s, jnp.arange(0, NUM_ITEMS + 1, R)).astype(jnp.int32)
    bu = jnp.pad(bu, (0, 240 - UCH - 1))
    bi = jnp.pad(bi, (0, 40 - ICH - 1))
    pad = EPAD - NE
    return (jnp.pad(su_s, (0, pad)), jnp.pad(du_s, (0, pad)), bu,
            jnp.pad(si_s, (0, pad)), jnp.pad(di_s, (0, pad)), bi), invd


def _rowscale_body(x_ref, d_ref, o_ref):
    o_ref[...] = x_ref[...] * d_ref[...]


def _rowscale(x, d):
    n = x.shape[0]
    return pl.pallas_call(
        _rowscale_body,
        grid=(n // 1000,),
        in_specs=[pl.BlockSpec((1000, EMB), lambda i: (i, 0)),
                  pl.BlockSpec((1000, 1), lambda i: (i, 0))],
        out_specs=pl.BlockSpec((1000, EMB), lambda i: (i, 0)),
        out_shape=jax.ShapeDtypeStruct((n, EMB), jnp.float32),
    )(x, d)


def _ii_body(zv, zt, xv, xt, sv, dv, wv, bv, st_, dt_, wt_, bt,
             out_v, out_t,
             bv_v, bt_v, dst_v, idx0, w0, l0, rows0, sem0,
             idx1, w1, l1, rows1, sem1, drain_v):
    c = lax.axis_index("c")
    s = lax.axis_index("s")
    wid = c * NS + s
    buf0 = (idx0, w0, l0, rows0, sem0)
    buf1 = (idx1, w1, l1, rows1, sem1)
    rows_v = rows0
    pltpu.sync_copy(bv, bv_v)
    pltpu.sync_copy(bt, bt_v)
    iota = jax.lax.iota(jnp.int32, L)

    def do_chunk(e0, e1, row0, ztab, xtab, e_src, e_dst, e_w, out):
        _zero_rows(drain_v, R2)
        _edge_blocks(e0, e1, row0, ztab, e_src, e_dst, e_w,
                     dst_v, buf0, buf1, drain_v, R2, iota)

        def mix_piece(base, n):
            pltpu.sync_copy(xtab.at[pl.ds(row0 + base, n)],
                            rows_v.at[pl.ds(0, n)])

            def body(r, _):
                for t in range(EMB // L):
                    sl = pl.ds(t * L, L)
                    drain_v[base + r, sl] = (ALPHA * drain_v[base + r, sl]
                                             + BETA * rows_v[r, sl])
                return 0
            lax.fori_loop(0, n, body, 0, unroll=False)

        mix_piece(0, B)
        mix_piece(B, R2 - B)
        pltpu.sync_copy(drain_v, out.at[pl.ds(row0, R2)])

    def slot(k, _):
        cid = k * NW + wid

        @pl.when(cid < VCH)
        def _():
            bv16 = bv_v[pl.ds(cid, L)]
            do_chunk(bv16[0], bv16[1], cid * R2, zv, xv, sv, dv, wv, out_v)

        @pl.when((cid >= VCH) & (cid < TOT2))
        def _():
            j = cid - VCH
            bv16 = bt_v[pl.ds(j, L)]
            do_chunk(bv16[0], bv16[1], j * R2, zt, xt, st_, dt_, wt_, out_t)

        return 0

    lax.fori_loop(0, SLOTS2, slot, 0, unroll=False)


def _make_ii():
    mesh = plsc.VectorSubcoreMesh(core_axis_name="c", subcore_axis_name="s",
                                  num_cores=NC, num_subcores=NS)
    return pl.kernel(
        _ii_body,
        out_type=(jax.ShapeDtypeStruct((NUM_ITEMS, EMB), jnp.float32),
                  jax.ShapeDtypeStruct((NUM_ITEMS, EMB), jnp.float32)),
        mesh=mesh,
        scratch_types=[
            pltpu.VMEM((40,), jnp.int32),
            pltpu.VMEM((40,), jnp.int32),
            pltpu.VMEM((B,), jnp.int32),
            pltpu.VMEM((B,), jnp.int32),
            pltpu.VMEM((B + L,), jnp.float32),
            pltpu.VMEM((B + L,), jnp.int32),
            pltpu.VMEM((B, EMB), jnp.float32),
            pltpu.SemaphoreType.DMA,
            pltpu.VMEM((B,), jnp.int32),
            pltpu.VMEM((B + L,), jnp.float32),
            pltpu.VMEM((B + L,), jnp.int32),
            pltpu.VMEM((B, EMB), jnp.float32),
            pltpu.SemaphoreType.DMA,
            pltpu.VMEM((R2, EMB), jnp.float32),
        ],
    )


def _ii_setup(src, dst, w):
    src32 = src.astype(jnp.int32)
    dst32 = dst.astype(jnp.int32)
    b = jnp.searchsorted(
        dst32, jnp.arange(0, NUM_ITEMS + 1, R2, dtype=jnp.int32)).astype(jnp.int32)
    b = jnp.pad(b, (0, 40 - VCH - 1))
    return (jnp.pad(src32, (0, 4 * B)), jnp.pad(dst32, (0, 4 * B)),
            jnp.pad(w, (0, 4 * B)), b)


def _combine_body(ue_ref, ie_ref, h1u_ref, h1i_ref, h2u_ref, h2i_ref,
                  vh_ref, th_ref, o_ref):
    i = pl.program_id(0)
    inv = 1.0 / (K + 1)

    @pl.when(i < NUM_USERS // 1000)
    def _():
        o_ref[...] = (ue_ref[...] + h1u_ref[...] + h2u_ref[...]) * inv

    @pl.when(i >= NUM_USERS // 1000)
    def _():
        o_ref[...] = ((ie_ref[...] + h1i_ref[...] + h2i_ref[...]) * inv
                      + vh_ref[...] + th_ref[...])


def kernel(user_emb, item_emb, v_feat, t_feat, vW1, vb1, vW2, vb2, vg, vbb,
           tW1, tb1, tW2, tb2, tg, tbb, v_src, v_dst, v_w,
           t_src, t_dst, t_w, g_src, g_dst, g_w):
    encode_v = _ffn_bn(v_feat, vW1, vb1, vW2, vb2, vg, vbb)
    encode_t = _ffn_bn(t_feat, tW1, tb1, tW2, tb2, tg, tbb)

    ii_prop = _make_ii()
    iargs = _ii_setup(v_src, v_dst, v_w) + _ii_setup(t_src, t_dst, t_w)
    zv, zt = encode_v, encode_t
    for _ in range(GCN_K):
        zv, zt = ii_prop(zv, zt, encode_v, encode_t, *iargs)

    prop = _make_prop()
    pargs, invd = _prop_setup(g_src, g_dst, g_w)
    invdu = invd[:NUM_USERS].reshape(-1, 1)
    invdi = invd[NUM_USERS:].reshape(-1, 1)
    dnu = jnp.sqrt(invdu)
    dni = jnp.sqrt(invdi)
    zu0 = _rowscale(user_emb, dnu)
    zi0 = _rowscale(item_emb, dni)
    acc1u, acc1i = prop(zu0, zi0, *pargs)
    h1u = _rowscale(acc1u, dnu)
    h1i = _rowscale(acc1i, dni)
    zu1 = _rowscale(acc1u, invdu)
    zi1 = _rowscale(acc1i, invdi)
    acc2u, acc2i = prop(zu1, zi1, *pargs)
    h2u = _rowscale(acc2u, dnu)
    h2i = _rowscale(acc2i, dni)

    nu_b = NUM_USERS // 1000
    node_h = pl.pallas_call(
        _combine_body,
        grid=(N_NODES // 1000,),
        in_specs=[
            pl.BlockSpec((1000, EMB), lambda i: (jnp.minimum(i, nu_b - 1), 0)),
            pl.BlockSpec((1000, EMB), lambda i: (jnp.maximum(i - nu_b, 0), 0)),
            pl.BlockSpec((1000, EMB), lambda i: (jnp.minimum(i, nu_b - 1), 0)),
            pl.BlockSpec((1000, EMB), lambda i: (jnp.maximum(i - nu_b, 0), 0)),
            pl.BlockSpec((1000, EMB), lambda i: (jnp.minimum(i, nu_b - 1), 0)),
            pl.BlockSpec((1000, EMB), lambda i: (jnp.maximum(i - nu_b, 0), 0)),
            pl.BlockSpec((1000, EMB), lambda i: (jnp.maximum(i - nu_b, 0), 0)),
            pl.BlockSpec((1000, EMB), lambda i: (jnp.maximum(i - nu_b, 0), 0)),
        ],
        out_specs=pl.BlockSpec((1000, EMB), lambda i: (i, 0)),
        out_shape=jax.ShapeDtypeStruct((N_NODES, EMB), jnp.float32),
    )(user_emb, item_emb, h1u, h1i, h2u, h2i, zv, zt)
    return node_h
